# baseline (speedup 1.0000x reference)
import functools
import math

import jax
import jax.numpy as jnp
from jax.experimental import pallas as pl
from jax.experimental.pallas import tpu as pltpu

F32 = jnp.float32
S, T, CM, CZ = 512, 256, 64, 128
CO = 32
CMUL = 128
CP, HP = 32, 4
CPWA, HPWA = 8, 8
EPS = 1e-5
VMEM_LIMIT = 100 * 1024 * 1024


def _ln(x, g, b):
    mu = jnp.mean(x, axis=-1, keepdims=True)
    var = jnp.mean(jnp.square(x - mu), axis=-1, keepdims=True)
    return (x - mu) * jax.lax.rsqrt(var + EPS) * g + b


def _dot(a, b):
    return jax.lax.dot_general(
        a, b, (((a.ndim - 1,), (0,)), ((), ())),
        preferred_element_type=F32)


def _full(shape):
    nd = len(shape)
    return pl.BlockSpec(shape, lambda *a: (0,) * nd)


def _cp(n):
    return pltpu.CompilerParams(
        dimension_semantics=("parallel",) * n,
        vmem_limit_bytes=VMEM_LIMIT,
    )


def _cp_arb(n):
    return pltpu.CompilerParams(
        dimension_semantics=("arbitrary",) * n,
        vmem_limit_bytes=VMEM_LIMIT,
    )


_OPM_BS = 64


def _opm_ab_body(m_ref, g_ref, b_ref, wab_ref, a_ref, bo_ref):
    h = _ln(m_ref[...].reshape(_OPM_BS * T, CM), g_ref[...], b_ref[...])
    ab = _dot(h, wab_ref[...])
    a_ref[...] = ab[:, :CO].reshape(_OPM_BS, T, CO)
    bo_ref[...] = ab[:, CO:].reshape(_OPM_BS, T, CO)


_OPM_BI = 32
_OPM_BJ = 32


def _opm_main_body(a_ref, b_ref, wo_ref, z_ref, o_ref):
    a = a_ref[...]
    b = b_ref[...]
    o4 = jax.lax.dot_general(
        a, b, (((0,), (0,)), ((), ())),
        preferred_element_type=F32)
    parts = [o4[:, c, :, :].reshape(_OPM_BI * _OPM_BJ, CO) for c in range(CO)]
    x = jnp.concatenate(parts, axis=-1)
    out = _dot(x, wo_ref[...])
    o_ref[...] = z_ref[...] + out.reshape(_OPM_BI, _OPM_BJ, CZ)


def _opm(m, z, p):
    wab = jnp.concatenate([p['Wa'], p['Wb']], axis=1)
    a, b = pl.pallas_call(
        _opm_ab_body,
        grid=(S // _OPM_BS,),
        in_specs=[
            pl.BlockSpec((_OPM_BS, T, CM), lambda i: (i, 0, 0)),
            _full((1, CM)), _full((1, CM)), _full((CM, 2 * CO)),
        ],
        out_specs=[
            pl.BlockSpec((_OPM_BS, T, CO), lambda i: (i, 0, 0)),
            pl.BlockSpec((_OPM_BS, T, CO), lambda i: (i, 0, 0)),
        ],
        out_shape=[
            jax.ShapeDtypeStruct((S, T, CO), F32),
            jax.ShapeDtypeStruct((S, T, CO), F32),
        ],
        compiler_params=_cp(1),
    )(m, p['ln_g'].reshape(1, CM), p['ln_b'].reshape(1, CM), wab)
    wo_scaled = p['Wo'] / float(S)
    return pl.pallas_call(
        _opm_main_body,
        grid=(T // _OPM_BI, T // _OPM_BJ),
        in_specs=[
            pl.BlockSpec((S, _OPM_BI, CO), lambda i, j: (0, i, 0)),
            pl.BlockSpec((S, _OPM_BJ, CO), lambda i, j: (0, j, 0)),
            _full((CO * CO, CZ)),
            pl.BlockSpec((_OPM_BI, _OPM_BJ, CZ), lambda i, j: (i, j, 0)),
        ],
        out_specs=pl.BlockSpec((_OPM_BI, _OPM_BJ, CZ), lambda i, j: (i, j, 0)),
        out_shape=jax.ShapeDtypeStruct((T, T, CZ), F32),
        compiler_params=_cp(2),
    )(a, b, wo_scaled, z)


_PWA_BI = 32
_PWA_BS = 64


def _pwa_w_body(z_ref, g_ref, b_ref, wb_ref, w_ref):
    h = _ln(z_ref[...].reshape(_PWA_BI * T, CZ), g_ref[...], b_ref[...])
    bb = _dot(h, wb_ref[...]).reshape(_PWA_BI, T, HPWA)
    bt = jnp.transpose(bb, (2, 0, 1))
    w_ref[...] = jax.nn.softmax(bt, axis=-1)


def _pwa_main_body(m_ref, w_ref, g_ref, b_ref, wv_ref, wg_ref, wo_ref, o_ref):
    x = m_ref[...]
    h2 = _ln(x.reshape(_PWA_BS * T, CM), g_ref[...], b_ref[...])
    v3 = _dot(h2, wv_ref[...]).reshape(_PWA_BS, T, HPWA * CPWA)
    g2 = jax.nn.sigmoid(_dot(h2, wg_ref[...]))
    vt = jnp.transpose(v3, (2, 0, 1))
    ys = []
    for hh in range(HPWA):
        wh = w_ref[hh]
        for cc in range(CPWA):
            xch = hh * CPWA + cc
            ys.append(jax.lax.dot_general(
                vt[xch], wh, (((1,), (1,)), ((), ())),
                preferred_element_type=F32))
    wvt = jnp.stack(ys, axis=0)
    o3 = jnp.transpose(wvt, (1, 2, 0))
    o2 = g2 * o3.reshape(_PWA_BS * T, HPWA * CPWA)
    o_ref[...] = x + _dot(o2, wo_ref[...]).reshape(_PWA_BS, T, CM)


def _pwa(m, z1, p):
    w8 = pl.pallas_call(
        _pwa_w_body,
        grid=(T // _PWA_BI,),
        in_specs=[
            pl.BlockSpec((_PWA_BI, T, CZ), lambda i: (i, 0, 0)),
            _full((1, CZ)), _full((1, CZ)), _full((CZ, HPWA)),
        ],
        out_specs=pl.BlockSpec((HPWA, _PWA_BI, T), lambda i: (0, i, 0)),
        out_shape=jax.ShapeDtypeStruct((HPWA, T, T), F32),
        compiler_params=_cp(1),
    )(z1, p['ln_z_g'].reshape(1, CZ), p['ln_z_b'].reshape(1, CZ), p['Wb'])
    return pl.pallas_call(
        _pwa_main_body,
        grid=(S // _PWA_BS,),
        in_specs=[
            pl.BlockSpec((_PWA_BS, T, CM), lambda i: (i, 0, 0)),
            _full((HPWA, T, T)),
            _full((1, CM)), _full((1, CM)),
            _full((CM, HPWA * CPWA)), _full((CM, HPWA * CPWA)),
            _full((HPWA * CPWA, CM)),
        ],
        out_specs=pl.BlockSpec((_PWA_BS, T, CM), lambda i: (i, 0, 0)),
        out_shape=jax.ShapeDtypeStruct((S, T, CM), F32),
        compiler_params=_cp(1),
    )(m, w8, p['ln_m_g'].reshape(1, CM), p['ln_m_b'].reshape(1, CM),
      p['Wv'], p['Wg'], p['Wo'])


def _trans_body(nrows, x_ref, g_ref, b_ref, wa_ref, wb_ref, wo_ref, o_ref):
    c = x_ref.shape[-1]
    x = x_ref[...].reshape(nrows, c)
    h = _ln(x, g_ref[...], b_ref[...])
    a = _dot(h, wa_ref[...])
    bb = _dot(h, wb_ref[...])
    o = _dot(jax.nn.silu(a) * bb, wo_ref[...])
    o_ref[...] = (x + o).reshape(x_ref.shape)


def _transition(x, p, bs):
    d0, d1, c = x.shape
    body = functools.partial(_trans_body, bs * d1)
    return pl.pallas_call(
        body,
        grid=(d0 // bs,),
        in_specs=[
            pl.BlockSpec((bs, d1, c), lambda i: (i, 0, 0)),
            _full((1, c)), _full((1, c)),
            _full((c, 4 * c)), _full((c, 4 * c)), _full((4 * c, c)),
        ],
        out_specs=pl.BlockSpec((bs, d1, c), lambda i: (i, 0, 0)),
        out_shape=jax.ShapeDtypeStruct(x.shape, F32),
        compiler_params=_cp(1),
    )(x, p['ln_g'].reshape(1, c), p['ln_b'].reshape(1, c),
      p['Wa'], p['Wb'], p['Wo'])


_TM_BP = 32
_TM_BI = 64
_TM_BJ = 64


def _tm_proj_body(z_ref, mask_ref, lg_ref, lb_ref, wag_ref, wap_ref,
                  wbg_ref, wbp_ref, at_ref, bt_ref):
    h2 = _ln(z_ref[...].reshape(_TM_BP * T, CZ), lg_ref[...], lb_ref[...])
    mk = mask_ref[...]
    a2 = jax.nn.sigmoid(_dot(h2, wag_ref[...])) * _dot(h2, wap_ref[...])
    b2 = jax.nn.sigmoid(_dot(h2, wbg_ref[...])) * _dot(h2, wbp_ref[...])
    at_ref[...] = jnp.transpose(mk * a2.reshape(_TM_BP, T, CMUL), (2, 0, 1))
    bt_ref[...] = jnp.transpose(mk * b2.reshape(_TM_BP, T, CMUL), (2, 0, 1))


def _tm_mul_body(at_ref, bt_ref, lg_ref, lb_ref, wo_ref,
                 lnin_g_ref, lnin_b_ref, wg_ref, z_ref, o_ref):
    at = at_ref[...]
    bt = bt_ref[...]
    x = jax.lax.dot_general(
        at, bt, (((2,), (2,)), ((0,), (0,))),
        preferred_element_type=F32)
    mu = jnp.mean(x, axis=0, keepdims=True)
    var = jnp.mean(jnp.square(x - mu), axis=0, keepdims=True)
    xn = (x - mu) * jax.lax.rsqrt(var + EPS) * lg_ref[...] + lb_ref[...]
    out = jax.lax.dot_general(
        xn, wo_ref[...], (((0,), (0,)), ((), ())),
        preferred_element_type=F32)
    zv = z_ref[...]
    h2 = _ln(zv.reshape(_TM_BI * _TM_BJ, CZ), lnin_g_ref[...],
             lnin_b_ref[...])
    g2 = jax.nn.sigmoid(_dot(h2, wg_ref[...])).reshape(_TM_BI, _TM_BJ, CZ)
    o_ref[...] = zv + g2 * out


def _tri_mul(z, mask, p, outgoing):
    src = z if outgoing else _transpose_z(z)
    msk = (mask if outgoing else jnp.swapaxes(mask, 0, 1))[:, :, None]
    at, bt = pl.pallas_call(
        _tm_proj_body,
        grid=(T // _TM_BP,),
        in_specs=[
            pl.BlockSpec((_TM_BP, T, CZ), lambda i: (i, 0, 0)),
            pl.BlockSpec((_TM_BP, T, 1), lambda i: (i, 0, 0)),
            _full((1, CZ)), _full((1, CZ)),
            _full((CZ, CMUL)), _full((CZ, CMUL)),
            _full((CZ, CMUL)), _full((CZ, CMUL)),
        ],
        out_specs=[
            pl.BlockSpec((CMUL, _TM_BP, T), lambda i: (0, i, 0)),
            pl.BlockSpec((CMUL, _TM_BP, T), lambda i: (0, i, 0)),
        ],
        out_shape=[
            jax.ShapeDtypeStruct((CMUL, T, T), F32),
            jax.ShapeDtypeStruct((CMUL, T, T), F32),
        ],
        compiler_params=_cp(1),
    )(src, msk, p['ln_in_g'].reshape(1, CZ), p['ln_in_b'].reshape(1, CZ),
      p['Wag'], p['Wap'], p['Wbg'], p['Wbp'])
    return pl.pallas_call(
        _tm_mul_body,
        grid=(T // _TM_BI, T // _TM_BJ),
        in_specs=[
            pl.BlockSpec((CMUL, _TM_BI, T), lambda i, j: (0, i, 0)),
            pl.BlockSpec((CMUL, _TM_BJ, T), lambda i, j: (0, j, 0)),
            _full((CMUL, 1, 1)), _full((CMUL, 1, 1)), _full((CMUL, CZ)),
            _full((1, CZ)), _full((1, CZ)), _full((CZ, CZ)),
            pl.BlockSpec((_TM_BI, _TM_BJ, CZ), lambda i, j: (i, j, 0)),
        ],
        out_specs=pl.BlockSpec((_TM_BI, _TM_BJ, CZ), lambda i, j: (i, j, 0)),
        out_shape=jax.ShapeDtypeStruct((T, T, CZ), F32),
        compiler_params=_cp(2),
    )(at, bt, p['ln_out_g'].reshape(CMUL, 1, 1),
      p['ln_out_b'].reshape(CMUL, 1, 1), p['Wo'],
      p['ln_in_g'].reshape(1, CZ), p['ln_in_b'].reshape(1, CZ), p['Wg'], z)


_TA_BP = 16
_TA_BI = 16


def _ta_proj_body(z_ref, lg_ref, lb_ref, wqkvg_ref, wbias_ref,
                  qkvg_ref, tb_ref):
    h2 = _ln(z_ref[...].reshape(_TA_BP * T, CZ), lg_ref[...], lb_ref[...])
    qkvg = _dot(h2, wqkvg_ref[...])
    qkvg_ref[...] = qkvg.reshape(_TA_BP, T, 4 * HP * CP)
    tb = _dot(h2, wbias_ref[...]).reshape(_TA_BP, T, HP)
    tb_ref[...] = jnp.transpose(tb, (2, 0, 1))


def _ta_att_body(qkvg_ref, tb_ref, mask_ref, wo_ref, z_ref, o_ref):
    qkvg = qkvg_ref[...]
    mb = (mask_ref[...] - 1.0) * 1e9
    scale = 1.0 / math.sqrt(CP)
    os_ = []
    for hh in range(HP):
        qh = qkvg[:, :, hh * CP:(hh + 1) * CP]
        kh = qkvg[:, :, 128 + hh * CP:128 + (hh + 1) * CP]
        vh = qkvg[:, :, 256 + hh * CP:256 + (hh + 1) * CP]
        lg = jax.lax.dot_general(
            qh, kh, (((2,), (2,)), ((0,), (0,))),
            preferred_element_type=F32) * scale
        lg = lg + tb_ref[hh][None, :, :] + mb[:, None, :]
        mx = jnp.max(lg, axis=-1, keepdims=True)
        e = jnp.exp(lg - mx)
        p = e / jnp.sum(e, axis=-1, keepdims=True)
        oh = jax.lax.dot_general(
            p, vh, (((2,), (1,)), ((0,), (0,))),
            preferred_element_type=F32)
        os_.append(oh)
    o = jnp.concatenate(os_, axis=-1).reshape(_TA_BI * T, HP * CP)
    g2 = jax.nn.sigmoid(qkvg[:, :, 384:]).reshape(_TA_BI * T, HP * CP)
    o2 = _dot(g2 * o, wo_ref[...])
    o_ref[...] = z_ref[...] + o2.reshape(_TA_BI, T, CZ)


def _tri_att(z, mask, p):
    wqkvg = jnp.concatenate([p['Wq'], p['Wk'], p['Wv'], p['Wg']], axis=1)
    qkvg, tb = pl.pallas_call(
        _ta_proj_body,
        grid=(T // _TA_BP,),
        in_specs=[
            pl.BlockSpec((_TA_BP, T, CZ), lambda i: (i, 0, 0)),
            _full((1, CZ)), _full((1, CZ)),
            _full((CZ, 4 * HP * CP)), _full((CZ, HP)),
        ],
        out_specs=[
            pl.BlockSpec((_TA_BP, T, 4 * HP * CP), lambda i: (i, 0, 0)),
            pl.BlockSpec((HP, _TA_BP, T), lambda i: (0, i, 0)),
        ],
        out_shape=[
            jax.ShapeDtypeStruct((T, T, 4 * HP * CP), F32),
            jax.ShapeDtypeStruct((HP, T, T), F32),
        ],
        compiler_params=_cp(1),
    )(z, p['ln_g'].reshape(1, CZ), p['ln_b'].reshape(1, CZ), wqkvg,
      p['Wbias'])
    return pl.pallas_call(
        _ta_att_body,
        grid=(T // _TA_BI,),
        in_specs=[
            pl.BlockSpec((_TA_BI, T, 4 * HP * CP), lambda i: (i, 0, 0)),
            _full((HP, T, T)),
            pl.BlockSpec((_TA_BI, T), lambda i: (i, 0)),
            _full((HP * CP, CZ)),
            pl.BlockSpec((_TA_BI, T, CZ), lambda i: (i, 0, 0)),
        ],
        out_specs=pl.BlockSpec((_TA_BI, T, CZ), lambda i: (i, 0, 0)),
        out_shape=jax.ShapeDtypeStruct((T, T, CZ), F32),
        compiler_params=_cp(1),
    )(qkvg, tb, mask, p['Wo'], z)


_TR_B = 64


def _transpose_body(x_ref, o_ref):
    o_ref[...] = jnp.transpose(x_ref[...], (1, 0, 2))


def _transpose_z(z):
    return pl.pallas_call(
        _transpose_body,
        grid=(T // _TR_B, T // _TR_B),
        in_specs=[pl.BlockSpec((_TR_B, _TR_B, CZ), lambda i, j: (j, i, 0))],
        out_specs=pl.BlockSpec((_TR_B, _TR_B, CZ), lambda i, j: (i, j, 0)),
        out_shape=jax.ShapeDtypeStruct((T, T, CZ), F32),
        compiler_params=_cp(2),
    )(z)


def kernel(m, z, pair_mask, params):
    z1 = _opm(m, z, params['opm'])
    m1 = _pwa(m, z1, params['pwa'])
    m2 = _transition(m1, params['msa_trans'], 32)
    z2 = _tri_mul(z1, pair_mask, params['tri_mul_out'], True)
    z3 = _tri_mul(z2, pair_mask, params['tri_mul_in'], False)
    z4 = _tri_att(z3, pair_mask, params['tri_att_start'])
    z4t = _transpose_z(z4)
    z5t = _tri_att(z4t, jnp.swapaxes(pair_mask, 0, 1), params['tri_att_end'])
    z5 = _transpose_z(z5t)
    z6 = _transition(z5, params['pair_trans'], 16)
    return m2, z6


# speedup vs baseline: 1.1120x; 1.1120x over previous
import functools
import math

import jax
import jax.numpy as jnp
from jax.experimental import pallas as pl
from jax.experimental.pallas import tpu as pltpu

F32 = jnp.float32
BF16 = jnp.bfloat16
S, T, CM, CZ = 512, 256, 64, 128
CO = 32
CMUL = 128
CP, HP = 32, 4
CPWA, HPWA = 8, 8
EPS = 1e-5
VMEM_LIMIT = 100 * 1024 * 1024


def _ln(x, g, b):
    mu = jnp.mean(x, axis=-1, keepdims=True)
    var = jnp.mean(jnp.square(x - mu), axis=-1, keepdims=True)
    return (x - mu) * jax.lax.rsqrt(var + EPS) * g + b


def _dot(a, b):
    return jax.lax.dot_general(
        a, b, (((a.ndim - 1,), (0,)), ((), ())),
        preferred_element_type=F32)


def _dotb(a, b):
    return jax.lax.dot_general(
        a.astype(BF16), b.astype(BF16), (((a.ndim - 1,), (0,)), ((), ())),
        preferred_element_type=F32)


def _full(shape):
    nd = len(shape)
    return pl.BlockSpec(shape, lambda *a: (0,) * nd)


def _cp(n):
    return pltpu.CompilerParams(
        dimension_semantics=("parallel",) * n,
        vmem_limit_bytes=VMEM_LIMIT,
    )


def _cp_arb(n):
    return pltpu.CompilerParams(
        dimension_semantics=("arbitrary",) * n,
        vmem_limit_bytes=VMEM_LIMIT,
    )


_OPM_BS = 64


def _opm_ab_body(m_ref, g_ref, b_ref, wab_ref, a_ref, bo_ref):
    h = _ln(m_ref[...].reshape(_OPM_BS * T, CM), g_ref[...], b_ref[...])
    ab = _dotb(h, wab_ref[...]).astype(BF16)
    a_ref[...] = ab[:, :CO].reshape(_OPM_BS, T, CO)
    bo_ref[...] = ab[:, CO:].reshape(_OPM_BS, T, CO)


_OPM_BI = 32
_OPM_BJ = 32


def _opm_main_body(a_ref, b_ref, wo_ref, z_ref, o_ref):
    a = a_ref[...]
    b = b_ref[...]
    o4 = jax.lax.dot_general(
        a, b, (((0,), (0,)), ((), ())),
        preferred_element_type=F32)
    parts = [o4[:, c, :, :].reshape(_OPM_BI * _OPM_BJ, CO) for c in range(CO)]
    x = jnp.concatenate(parts, axis=-1)
    out = _dotb(x, wo_ref[...])
    o_ref[...] = z_ref[...] + out.reshape(_OPM_BI, _OPM_BJ, CZ)


def _opm(m, z, p):
    wab = jnp.concatenate([p['Wa'], p['Wb']], axis=1)
    a, b = pl.pallas_call(
        _opm_ab_body,
        grid=(S // _OPM_BS,),
        in_specs=[
            pl.BlockSpec((_OPM_BS, T, CM), lambda i: (i, 0, 0)),
            _full((1, CM)), _full((1, CM)), _full((CM, 2 * CO)),
        ],
        out_specs=[
            pl.BlockSpec((_OPM_BS, T, CO), lambda i: (i, 0, 0)),
            pl.BlockSpec((_OPM_BS, T, CO), lambda i: (i, 0, 0)),
        ],
        out_shape=[
            jax.ShapeDtypeStruct((S, T, CO), BF16),
            jax.ShapeDtypeStruct((S, T, CO), BF16),
        ],
        compiler_params=_cp(1),
    )(m, p['ln_g'].reshape(1, CM), p['ln_b'].reshape(1, CM), wab)
    wo_scaled = p['Wo'] / float(S)
    return pl.pallas_call(
        _opm_main_body,
        grid=(T // _OPM_BI, T // _OPM_BJ),
        in_specs=[
            pl.BlockSpec((S, _OPM_BI, CO), lambda i, j: (0, i, 0)),
            pl.BlockSpec((S, _OPM_BJ, CO), lambda i, j: (0, j, 0)),
            _full((CO * CO, CZ)),
            pl.BlockSpec((_OPM_BI, _OPM_BJ, CZ), lambda i, j: (i, j, 0)),
        ],
        out_specs=pl.BlockSpec((_OPM_BI, _OPM_BJ, CZ), lambda i, j: (i, j, 0)),
        out_shape=jax.ShapeDtypeStruct((T, T, CZ), F32),
        compiler_params=_cp(2),
    )(a, b, wo_scaled.astype(BF16), z)


_PWA_BI = 32
_PWA_BS = 64


def _pwa_w_body(z_ref, g_ref, b_ref, wb_ref, w_ref):
    h = _ln(z_ref[...].reshape(_PWA_BI * T, CZ), g_ref[...], b_ref[...])
    bb = _dotb(h, wb_ref[...]).reshape(_PWA_BI, T, HPWA)
    bt = jnp.transpose(bb, (2, 0, 1))
    w_ref[...] = jax.nn.softmax(bt, axis=-1).astype(BF16)


def _pwa_main_body(m_ref, w_ref, g_ref, b_ref, wv_ref, wg_ref, wo_ref, o_ref):
    x = m_ref[...]
    h2 = _ln(x.reshape(_PWA_BS * T, CM), g_ref[...], b_ref[...])
    v3 = _dotb(h2, wv_ref[...]).astype(BF16).reshape(_PWA_BS, T, HPWA * CPWA)
    g2 = jax.nn.sigmoid(_dot(h2, wg_ref[...]))
    vt = jnp.transpose(v3, (2, 0, 1))
    ys = []
    for hh in range(HPWA):
        wh = w_ref[hh]
        for cc in range(CPWA):
            xch = hh * CPWA + cc
            ys.append(jax.lax.dot_general(
                vt[xch], wh, (((1,), (1,)), ((), ())),
                preferred_element_type=F32))
    wvt = jnp.stack(ys, axis=0)
    o3 = jnp.transpose(wvt, (1, 2, 0))
    o2 = g2 * o3.reshape(_PWA_BS * T, HPWA * CPWA)
    o_ref[...] = x + _dotb(o2, wo_ref[...]).reshape(_PWA_BS, T, CM)


def _pwa(m, z1, p):
    w8 = pl.pallas_call(
        _pwa_w_body,
        grid=(T // _PWA_BI,),
        in_specs=[
            pl.BlockSpec((_PWA_BI, T, CZ), lambda i: (i, 0, 0)),
            _full((1, CZ)), _full((1, CZ)), _full((CZ, HPWA)),
        ],
        out_specs=pl.BlockSpec((HPWA, _PWA_BI, T), lambda i: (0, i, 0)),
        out_shape=jax.ShapeDtypeStruct((HPWA, T, T), BF16),
        compiler_params=_cp(1),
    )(z1, p['ln_z_g'].reshape(1, CZ), p['ln_z_b'].reshape(1, CZ), p['Wb'])
    return pl.pallas_call(
        _pwa_main_body,
        grid=(S // _PWA_BS,),
        in_specs=[
            pl.BlockSpec((_PWA_BS, T, CM), lambda i: (i, 0, 0)),
            _full((HPWA, T, T)),
            _full((1, CM)), _full((1, CM)),
            _full((CM, HPWA * CPWA)), _full((CM, HPWA * CPWA)),
            _full((HPWA * CPWA, CM)),
        ],
        out_specs=pl.BlockSpec((_PWA_BS, T, CM), lambda i: (i, 0, 0)),
        out_shape=jax.ShapeDtypeStruct((S, T, CM), F32),
        compiler_params=_cp(1),
    )(m, w8, p['ln_m_g'].reshape(1, CM), p['ln_m_b'].reshape(1, CM),
      p['Wv'], p['Wg'], p['Wo'])


def _trans_body(nrows, x_ref, g_ref, b_ref, wa_ref, wb_ref, wo_ref, o_ref):
    c = x_ref.shape[-1]
    x = x_ref[...].reshape(nrows, c)
    h = _ln(x, g_ref[...], b_ref[...]).astype(BF16)
    a = _dot(h, wa_ref[...].astype(BF16))
    bb = _dot(h, wb_ref[...].astype(BF16))
    o = _dotb(jax.nn.silu(a) * bb, wo_ref[...])
    o_ref[...] = (x + o).reshape(x_ref.shape)


def _transition(x, p, bs):
    d0, d1, c = x.shape
    body = functools.partial(_trans_body, bs * d1)
    return pl.pallas_call(
        body,
        grid=(d0 // bs,),
        in_specs=[
            pl.BlockSpec((bs, d1, c), lambda i: (i, 0, 0)),
            _full((1, c)), _full((1, c)),
            _full((c, 4 * c)), _full((c, 4 * c)), _full((4 * c, c)),
        ],
        out_specs=pl.BlockSpec((bs, d1, c), lambda i: (i, 0, 0)),
        out_shape=jax.ShapeDtypeStruct(x.shape, F32),
        compiler_params=_cp(1),
    )(x, p['ln_g'].reshape(1, c), p['ln_b'].reshape(1, c),
      p['Wa'], p['Wb'], p['Wo'])


_TM_BP = 32
_TM_BI = 64
_TM_BJ = 64


def _tm_proj_body(z_ref, mask_ref, lg_ref, lb_ref, wag_ref, wap_ref,
                  wbg_ref, wbp_ref, at_ref, bt_ref):
    h2 = _ln(z_ref[...].reshape(_TM_BP * T, CZ), lg_ref[...], lb_ref[...])
    mk = mask_ref[...]
    a2 = jax.nn.sigmoid(_dot(h2, wag_ref[...])) * _dot(h2, wap_ref[...])
    b2 = jax.nn.sigmoid(_dot(h2, wbg_ref[...])) * _dot(h2, wbp_ref[...])
    at_ref[...] = jnp.transpose(
        (mk * a2.reshape(_TM_BP, T, CMUL)).astype(BF16), (2, 0, 1))
    bt_ref[...] = jnp.transpose(
        (mk * b2.reshape(_TM_BP, T, CMUL)).astype(BF16), (2, 0, 1))


def _tm_mul_body(at_ref, bt_ref, lg_ref, lb_ref, wo_ref,
                 lnin_g_ref, lnin_b_ref, wg_ref, z_ref, o_ref):
    at = at_ref[...]
    bt = bt_ref[...]
    x = jax.lax.dot_general(
        at, bt, (((2,), (2,)), ((0,), (0,))),
        preferred_element_type=F32)
    mu = jnp.mean(x, axis=0, keepdims=True)
    var = jnp.mean(jnp.square(x - mu), axis=0, keepdims=True)
    xn = (x - mu) * jax.lax.rsqrt(var + EPS) * lg_ref[...] + lb_ref[...]
    out = jax.lax.dot_general(
        xn.astype(BF16), wo_ref[...].astype(BF16), (((0,), (0,)), ((), ())),
        preferred_element_type=F32)
    zv = z_ref[...]
    h2 = _ln(zv.reshape(_TM_BI * _TM_BJ, CZ), lnin_g_ref[...],
             lnin_b_ref[...])
    g2 = jax.nn.sigmoid(_dotb(h2, wg_ref[...])).reshape(_TM_BI, _TM_BJ, CZ)
    o_ref[...] = zv + g2 * out


def _tri_mul(z, mask, p, outgoing):
    src = z if outgoing else _transpose_z(z)
    msk = (mask if outgoing else jnp.swapaxes(mask, 0, 1))[:, :, None]
    at, bt = pl.pallas_call(
        _tm_proj_body,
        grid=(T // _TM_BP,),
        in_specs=[
            pl.BlockSpec((_TM_BP, T, CZ), lambda i: (i, 0, 0)),
            pl.BlockSpec((_TM_BP, T, 1), lambda i: (i, 0, 0)),
            _full((1, CZ)), _full((1, CZ)),
            _full((CZ, CMUL)), _full((CZ, CMUL)),
            _full((CZ, CMUL)), _full((CZ, CMUL)),
        ],
        out_specs=[
            pl.BlockSpec((CMUL, _TM_BP, T), lambda i: (0, i, 0)),
            pl.BlockSpec((CMUL, _TM_BP, T), lambda i: (0, i, 0)),
        ],
        out_shape=[
            jax.ShapeDtypeStruct((CMUL, T, T), BF16),
            jax.ShapeDtypeStruct((CMUL, T, T), BF16),
        ],
        compiler_params=_cp(1),
    )(src, msk, p['ln_in_g'].reshape(1, CZ), p['ln_in_b'].reshape(1, CZ),
      p['Wag'], p['Wap'], p['Wbg'], p['Wbp'])
    return pl.pallas_call(
        _tm_mul_body,
        grid=(T // _TM_BI, T // _TM_BJ),
        in_specs=[
            pl.BlockSpec((CMUL, _TM_BI, T), lambda i, j: (0, i, 0)),
            pl.BlockSpec((CMUL, _TM_BJ, T), lambda i, j: (0, j, 0)),
            _full((CMUL, 1, 1)), _full((CMUL, 1, 1)), _full((CMUL, CZ)),
            _full((1, CZ)), _full((1, CZ)), _full((CZ, CZ)),
            pl.BlockSpec((_TM_BI, _TM_BJ, CZ), lambda i, j: (i, j, 0)),
        ],
        out_specs=pl.BlockSpec((_TM_BI, _TM_BJ, CZ), lambda i, j: (i, j, 0)),
        out_shape=jax.ShapeDtypeStruct((T, T, CZ), F32),
        compiler_params=_cp(2),
    )(at, bt, p['ln_out_g'].reshape(CMUL, 1, 1),
      p['ln_out_b'].reshape(CMUL, 1, 1), p['Wo'],
      p['ln_in_g'].reshape(1, CZ), p['ln_in_b'].reshape(1, CZ), p['Wg'], z)


_TA_BP = 16
_TA_BI = 16


def _ta_proj_body(z_ref, lg_ref, lb_ref, wqkvg_ref, wbias_ref,
                  qkvg_ref, tb_ref):
    h2 = _ln(z_ref[...].reshape(_TA_BP * T, CZ), lg_ref[...], lb_ref[...])
    qkvg = _dotb(h2, wqkvg_ref[...]).astype(BF16)
    qkvg_ref[...] = qkvg.reshape(_TA_BP, T, 4 * HP * CP)
    tb = _dotb(h2, wbias_ref[...]).reshape(_TA_BP, T, HP)
    tb_ref[...] = jnp.transpose(tb, (2, 0, 1))


def _ta_att_body(qkvg_ref, tb_ref, mask_ref, wo_ref, z_ref, o_ref):
    qkvg = qkvg_ref[...]
    mb = (mask_ref[...] - 1.0) * 1e9
    scale = 1.0 / math.sqrt(CP)
    os_ = []
    for hh in range(HP):
        qh = qkvg[:, :, hh * CP:(hh + 1) * CP]
        kh = qkvg[:, :, 128 + hh * CP:128 + (hh + 1) * CP]
        vh = qkvg[:, :, 256 + hh * CP:256 + (hh + 1) * CP]
        lg = jax.lax.dot_general(
            qh, kh, (((2,), (2,)), ((0,), (0,))),
            preferred_element_type=F32) * scale
        lg = lg + tb_ref[hh][None, :, :] + mb[:, None, :]
        mx = jnp.max(lg, axis=-1, keepdims=True)
        e = jnp.exp(lg - mx)
        p = (e / jnp.sum(e, axis=-1, keepdims=True)).astype(BF16)
        oh = jax.lax.dot_general(
            p, vh, (((2,), (1,)), ((0,), (0,))),
            preferred_element_type=F32)
        os_.append(oh)
    o = jnp.concatenate(os_, axis=-1).reshape(_TA_BI * T, HP * CP)
    g2 = jax.nn.sigmoid(
        qkvg[:, :, 384:].astype(F32)).reshape(_TA_BI * T, HP * CP)
    o2 = _dotb(g2 * o, wo_ref[...])
    o_ref[...] = z_ref[...] + o2.reshape(_TA_BI, T, CZ)


def _tri_att(z, mask, p):
    wqkvg = jnp.concatenate([p['Wq'], p['Wk'], p['Wv'], p['Wg']], axis=1)
    qkvg, tb = pl.pallas_call(
        _ta_proj_body,
        grid=(T // _TA_BP,),
        in_specs=[
            pl.BlockSpec((_TA_BP, T, CZ), lambda i: (i, 0, 0)),
            _full((1, CZ)), _full((1, CZ)),
            _full((CZ, 4 * HP * CP)), _full((CZ, HP)),
        ],
        out_specs=[
            pl.BlockSpec((_TA_BP, T, 4 * HP * CP), lambda i: (i, 0, 0)),
            pl.BlockSpec((HP, _TA_BP, T), lambda i: (0, i, 0)),
        ],
        out_shape=[
            jax.ShapeDtypeStruct((T, T, 4 * HP * CP), BF16),
            jax.ShapeDtypeStruct((HP, T, T), F32),
        ],
        compiler_params=_cp(1),
    )(z, p['ln_g'].reshape(1, CZ), p['ln_b'].reshape(1, CZ), wqkvg,
      p['Wbias'])
    return pl.pallas_call(
        _ta_att_body,
        grid=(T // _TA_BI,),
        in_specs=[
            pl.BlockSpec((_TA_BI, T, 4 * HP * CP), lambda i: (i, 0, 0)),
            _full((HP, T, T)),
            pl.BlockSpec((_TA_BI, T), lambda i: (i, 0)),
            _full((HP * CP, CZ)),
            pl.BlockSpec((_TA_BI, T, CZ), lambda i: (i, 0, 0)),
        ],
        out_specs=pl.BlockSpec((_TA_BI, T, CZ), lambda i: (i, 0, 0)),
        out_shape=jax.ShapeDtypeStruct((T, T, CZ), F32),
        compiler_params=_cp(1),
    )(qkvg, tb, mask, p['Wo'], z)


_TR_B = 64


def _transpose_body(x_ref, o_ref):
    o_ref[...] = jnp.transpose(x_ref[...], (1, 0, 2))


def _transpose_z(z):
    return pl.pallas_call(
        _transpose_body,
        grid=(T // _TR_B, T // _TR_B),
        in_specs=[pl.BlockSpec((_TR_B, _TR_B, CZ), lambda i, j: (j, i, 0))],
        out_specs=pl.BlockSpec((_TR_B, _TR_B, CZ), lambda i, j: (i, j, 0)),
        out_shape=jax.ShapeDtypeStruct((T, T, CZ), F32),
        compiler_params=_cp(2),
    )(z)


def kernel(m, z, pair_mask, params):
    z1 = _opm(m, z, params['opm'])
    m1 = _pwa(m, z1, params['pwa'])
    m2 = _transition(m1, params['msa_trans'], 32)
    z2 = _tri_mul(z1, pair_mask, params['tri_mul_out'], True)
    z3 = _tri_mul(z2, pair_mask, params['tri_mul_in'], False)
    z4 = _tri_att(z3, pair_mask, params['tri_att_start'])
    z4t = _transpose_z(z4)
    z5t = _tri_att(z4t, jnp.swapaxes(pair_mask, 0, 1), params['tri_att_end'])
    z5 = _transpose_z(z5t)
    z6 = _transition(z5, params['pair_trans'], 16)
    return m2, z6
